# baseline (speedup 1.0000x reference)
import jax
import jax.numpy as jnp
from jax import lax
from jax.experimental import pallas as pl
from jax.experimental.pallas import tpu as pltpu
from jax.experimental.pallas import tpu_sc as plsc

_EMB = 64
_SCALE = 8.0

_NC = 2
_NS = 16
_NW = _NC * _NS

_CHUNK = 512
_DMA_ROWS = 128


def _emb_body(tokens_hbm, table_hbm, out_hbm, idx_v, rows_v, sem):
    n_tok = tokens_hbm.shape[0]
    per_w = n_tok // _NW
    n_chunks = per_w // _CHUNK
    wid = lax.axis_index("s") * _NC + lax.axis_index("c")
    base = wid * per_w

    def chunk_body(g, carry):
        off = base + g * _CHUNK
        pltpu.sync_copy(tokens_hbm.at[pl.ds(off, _CHUNK)], idx_v)
        copies = [
            pltpu.async_copy(
                table_hbm.at[idx_v.at[pl.ds(j * _DMA_ROWS, _DMA_ROWS)]],
                rows_v.at[pl.ds(j * _DMA_ROWS, _DMA_ROWS)],
                sem,
            )
            for j in range(_CHUNK // _DMA_ROWS)
        ]
        for cp in copies:
            cp.wait()

        def scale_row(r, c2):
            for c in range(_EMB // 16):
                sl = pl.ds(c * 16, 16)
                rows_v[r, sl] = rows_v[r, sl] * _SCALE
            return c2

        lax.fori_loop(0, _CHUNK, scale_row, 0)
        pltpu.sync_copy(rows_v, out_hbm.at[pl.ds(off, _CHUNK)])
        return carry

    lax.fori_loop(0, n_chunks, chunk_body, 0)


def kernel(tokens, table):
    b, l = tokens.shape
    n_tok = b * l
    flat = tokens.reshape(n_tok)
    mesh = plsc.VectorSubcoreMesh(core_axis_name="c", subcore_axis_name="s")
    out = pl.kernel(
        _emb_body,
        out_type=jax.ShapeDtypeStruct((n_tok, _EMB), jnp.float32),
        mesh=mesh,
        scratch_types=[
            pltpu.VMEM((_CHUNK,), jnp.int32),
            pltpu.VMEM((_CHUNK, _EMB), jnp.float32),
            pltpu.SemaphoreType.DMA,
        ],
        compiler_params=pltpu.CompilerParams(use_tc_tiling_on_sc=False),
    )(flat, table)
    return out.reshape(b, l, _EMB)


# speedup vs baseline: 1.1329x; 1.1329x over previous
import jax
import jax.numpy as jnp
from jax import lax
from jax.experimental import pallas as pl
from jax.experimental.pallas import tpu as pltpu
from jax.experimental.pallas import tpu_sc as plsc

_EMB = 64
_SCALE = 8.0

_NC = 2
_NS = 16
_NW = _NC * _NS

_CHUNK = 640
_DMA_ROWS = 128
_K = _CHUNK // _DMA_ROWS


def _emb_body(tokens_hbm, table_hbm, out_hbm,
              idx_all, rows0, rows1, gsem0, gsem1, osem0, osem1):
    n_tok = tokens_hbm.shape[0]
    per_w = n_tok // _NW
    n_chunks = per_w // _CHUNK
    n2 = n_chunks // 2
    wid = lax.axis_index("s") * _NC + lax.axis_index("c")
    base = wid * per_w

    rows = (rows0, rows1)
    gsem = (gsem0, gsem1)
    osem = (osem0, osem1)

    pltpu.sync_copy(tokens_hbm.at[pl.ds(base, per_w)], idx_all)

    def start_gather(gg, b):
        for j in range(_K):
            pltpu.async_copy(
                table_hbm.at[idx_all.at[pl.ds(gg * _CHUNK + j * _DMA_ROWS,
                                              _DMA_ROWS)]],
                rows[b].at[pl.ds(j * _DMA_ROWS, _DMA_ROWS)],
                gsem[b],
            )

    def wait_gather(b):
        pltpu.make_async_copy(table_hbm.at[pl.ds(0, _CHUNK)], rows[b],
                              gsem[b]).wait()

    def start_outcopy(gg, b):
        pltpu.async_copy(rows[b], out_hbm.at[pl.ds(base + gg * _CHUNK, _CHUNK)],
                         osem[b])

    def wait_outcopy(b):
        pltpu.make_async_copy(rows[b], out_hbm.at[pl.ds(base, _CHUNK)],
                              osem[b]).wait()

    def scale(b):
        buf = rows[b]

        def srow(i, c):
            r = i * 4
            for dr in range(4):
                for c4 in range(_EMB // 16):
                    sl = pl.ds(c4 * 16, 16)
                    buf[r + dr, sl] = buf[r + dr, sl] * _SCALE
            return c

        lax.fori_loop(0, _CHUNK // 4, srow, 0)

    start_gather(0, 0)

    def outer(g2, carry):
        gg0 = 2 * g2

        @pl.when(g2 > 0)
        def _():
            wait_outcopy(1)
        start_gather(gg0 + 1, 1)
        wait_gather(0)
        scale(0)
        start_outcopy(gg0, 0)

        @pl.when(g2 < n2 - 1)
        def _():
            wait_outcopy(0)
            start_gather(gg0 + 2, 0)
        wait_gather(1)
        scale(1)
        start_outcopy(gg0 + 1, 1)
        return carry

    lax.fori_loop(0, n2, outer, 0)
    wait_outcopy(0)
    wait_outcopy(1)


def kernel(tokens, table):
    b, l = tokens.shape
    n_tok = b * l
    flat = tokens.reshape(n_tok)
    per_w = n_tok // _NW
    mesh = plsc.VectorSubcoreMesh(core_axis_name="c", subcore_axis_name="s")
    out = pl.kernel(
        _emb_body,
        out_type=jax.ShapeDtypeStruct((n_tok, _EMB), jnp.float32),
        mesh=mesh,
        scratch_types=[
            pltpu.VMEM((per_w,), jnp.int32),
            pltpu.VMEM((_CHUNK, _EMB), jnp.float32),
            pltpu.VMEM((_CHUNK, _EMB), jnp.float32),
            pltpu.SemaphoreType.DMA,
            pltpu.SemaphoreType.DMA,
            pltpu.SemaphoreType.DMA,
            pltpu.SemaphoreType.DMA,
        ],
        compiler_params=pltpu.CompilerParams(use_tc_tiling_on_sc=False),
    )(flat, table)
    return out.reshape(b, l, _EMB)
